# baseline (speedup 1.0000x reference)
import functools

import jax
import jax.numpy as jnp
from jax import lax
from jax.experimental import pallas as pl
from jax.experimental.pallas import tpu as pltpu
from jax.experimental.pallas import tpu_sc as plsc

NC, NS = 2, 16
NW = NC * NS
LN = 128
JC = 25


@functools.lru_cache(maxsize=None)
def _make_kernel(nrows: int, ncols: int, vocab: int, embed_dim: int):
    mesh = plsc.VectorSubcoreMesh(core_axis_name="c", subcore_axis_name="s")
    nit = nrows // LN
    nit_w = nit // NW
    njc = ncols // JC
    kt = embed_dim // 8

    @functools.partial(
        pl.kernel,
        out_type=jax.ShapeDtypeStruct((ncols, kt, nit, 8, LN), jnp.float32),
        mesh=mesh,
        compiler_params=pltpu.CompilerParams(
            use_tc_tiling_on_sc=False, needs_layout_passes=False
        ),
        scratch_types=[
            pltpu.VMEM((embed_dim * vocab * 16,), jnp.float32),
            pltpu.VMEM((2, JC, LN), jnp.int32),
            pltpu.VMEM((2, JC, kt, 8, LN), jnp.float32),
            pltpu.SemaphoreType.DMA,
            pltpu.SemaphoreType.DMA,
        ],
    )
    def kern(ids_t_hbm, table_hbm, out_hbm, table_v, idx_v, out_v, sem_o, sem_i):
        cid = lax.axis_index("c")
        sid = lax.axis_index("s")
        wid = sid * NC + cid
        nchunks = nit_w * njc

        pltpu.sync_copy(table_hbm, table_v)

        lane = lax.iota(jnp.int32, 16)
        full = jnp.ones((16,), dtype=jnp.bool_)

        def out_slice(c):
            it = wid * nit_w + c // njc
            j0 = (c % njc) * JC
            return out_hbm.at[pl.ds(j0, JC), :, it]

        def ids_slice(c):
            cc = jnp.minimum(c, nchunks - 1)
            it = wid * nit_w + cc // njc
            j0 = (cc % njc) * JC
            return ids_t_hbm.at[pl.ds(j0, JC), pl.ds(it * LN, LN)]

        def idx_start(c, b):
            pltpu.async_copy(ids_slice(c), idx_v.at[b], sem_i)

        def idx_wait(c, b):
            pltpu.make_async_copy(ids_slice(c), idx_v.at[b], sem_i).wait()

        def compute_chunk(c, b):
            @plsc.parallel_loop(0, JC, unroll=4)
            def j_loop(jj):
                for g0 in range(0, LN // 16, 2):
                    bases = [
                        idx_v[b, jj, pl.ds((g0 + d) * 16, 16)] * 16 + lane
                        for d in range(2)
                    ]
                    for k in range(embed_dim):
                        for d in range(2):
                            v = plsc.load_gather(
                                table_v, [bases[d] + k * vocab * 16]
                            )
                            plsc.store_compressed(
                                out_v.at[
                                    b, jj, k // 8, k % 8,
                                    pl.ds((g0 + d) * 16, 16),
                                ],
                                v,
                                mask=full,
                            )

        def store_start(c, b):
            pltpu.async_copy(out_v.at[b], out_slice(c), sem_o)

        def store_wait(c, b):
            pltpu.make_async_copy(out_v.at[b], out_slice(c), sem_o).wait()

        idx_start(0, 0)
        idx_wait(0, 0)
        idx_start(1, 1)
        compute_chunk(0, 0)
        store_start(0, 0)
        idx_wait(1, 1)
        idx_start(2, 0)
        compute_chunk(1, 1)
        store_start(1, 1)

        @pl.loop(2, nchunks, step=2)
        def chunk_loop(c0):
            for b in range(2):
                c = c0 + b
                idx_wait(c, b)

                @pl.when(c + 1 < nchunks)
                def _():
                    idx_start(c + 1, 1 - b)

                store_wait(c - 2, b)
                compute_chunk(c, b)
                store_start(c, b)

        store_wait(nchunks - 2, 0)
        store_wait(nchunks - 1, 1)

    return kern


def kernel(ids, embed):
    nrows, ncols = ids.shape
    vocab, embed_dim = embed.shape
    ids_t = jnp.transpose(ids.astype(jnp.int32))
    table_rep = jnp.broadcast_to(
        jnp.transpose(embed)[:, :, None], (embed_dim, vocab, 16)
    ).reshape(-1)
    a = _make_kernel(nrows, ncols, vocab, embed_dim)(ids_t, table_rep)
    return a.transpose(2, 4, 0, 1, 3).reshape(nrows, ncols, embed_dim)


# speedup vs baseline: 1.3754x; 1.3754x over previous
import functools

import jax
import jax.numpy as jnp
from jax import lax
from jax.experimental import pallas as pl
from jax.experimental.pallas import tpu as pltpu
from jax.experimental.pallas import tpu_sc as plsc

NC, NS = 2, 16
NW = NC * NS
LN = 128
JC = 25


@functools.lru_cache(maxsize=None)
def _make_kernel(nrows: int, ncols: int, vocab: int, embed_dim: int):
    mesh = plsc.VectorSubcoreMesh(core_axis_name="c", subcore_axis_name="s")
    nit = nrows // LN
    nit_w = nit // NW
    njc = ncols // JC
    kt = embed_dim // 8

    @functools.partial(
        pl.kernel,
        out_type=jax.ShapeDtypeStruct((ncols, kt, nit, 8, LN), jnp.float32),
        mesh=mesh,
        compiler_params=pltpu.CompilerParams(
            use_tc_tiling_on_sc=False, needs_layout_passes=False
        ),
        scratch_types=[
            pltpu.VMEM((embed_dim * vocab * 16,), jnp.float32),
            pltpu.VMEM((2, JC, LN), jnp.int32),
            pltpu.VMEM((2, JC, kt, 8, LN), jnp.float32),
            pltpu.SemaphoreType.DMA,
            pltpu.SemaphoreType.DMA,
        ],
    )
    def kern(ids_t_hbm, table_hbm, out_hbm, table_v, idx_v, out_v, sem_o, sem_i):
        cid = lax.axis_index("c")
        sid = lax.axis_index("s")
        wid = sid * NC + cid
        nchunks = nit_w * njc

        pltpu.sync_copy(table_hbm, table_v)

        lane = lax.iota(jnp.int32, 16)
        full = jnp.ones((16,), dtype=jnp.bool_)

        def out_slice(c):
            it = wid * nit_w + c // njc
            j0 = (c % njc) * JC
            return out_hbm.at[pl.ds(j0, JC), :, it]

        def ids_slice(c):
            cc = jnp.minimum(c, nchunks - 1)
            it = wid * nit_w + cc // njc
            j0 = (cc % njc) * JC
            return ids_t_hbm.at[pl.ds(j0, JC), pl.ds(it * LN, LN)]

        def idx_start(c, b):
            pltpu.async_copy(ids_slice(c), idx_v.at[b], sem_i)

        def idx_wait(c, b):
            pltpu.make_async_copy(ids_slice(c), idx_v.at[b], sem_i).wait()

        def compute_chunk(c, b):
            @plsc.parallel_loop(0, JC, unroll=5)
            def j_loop(jj):
                bases = [
                    idx_v[b, jj, pl.ds(g * 16, 16)] * 16 + lane
                    for g in range(LN // 16)
                ]
                for k in range(embed_dim):
                    for g in range(LN // 16):
                        v = plsc.load_gather(
                            table_v, [bases[g] + k * vocab * 16]
                        )
                        out_v[b, jj, k // 8, k % 8, pl.ds(g * 16, 16)] = v

        def store_start(c, b):
            pltpu.async_copy(out_v.at[b], out_slice(c), sem_o)

        def store_wait(c, b):
            pltpu.make_async_copy(out_v.at[b], out_slice(c), sem_o).wait()

        idx_start(0, 0)
        idx_wait(0, 0)
        idx_start(1, 1)
        compute_chunk(0, 0)
        store_start(0, 0)
        idx_wait(1, 1)
        idx_start(2, 0)
        compute_chunk(1, 1)
        store_start(1, 1)

        @pl.loop(2, nchunks, step=2)
        def chunk_loop(c0):
            for b in range(2):
                c = c0 + b
                idx_wait(c, b)

                @pl.when(c + 1 < nchunks)
                def _():
                    idx_start(c + 1, 1 - b)

                store_wait(c - 2, b)
                compute_chunk(c, b)
                store_start(c, b)

        store_wait(nchunks - 2, 0)
        store_wait(nchunks - 1, 1)

    return kern


def kernel(ids, embed):
    nrows, ncols = ids.shape
    vocab, embed_dim = embed.shape
    ids_t = jnp.transpose(ids.astype(jnp.int32))
    table_rep = jnp.broadcast_to(
        jnp.transpose(embed)[:, :, None], (embed_dim, vocab, 16)
    ).reshape(-1)
    a = _make_kernel(nrows, ncols, vocab, embed_dim)(ids_t, table_rep)
    return a.transpose(2, 4, 0, 1, 3).reshape(nrows, ncols, embed_dim)


# speedup vs baseline: 2.7118x; 1.9716x over previous
import functools

import jax
import jax.numpy as jnp
from jax import lax
from jax.experimental import pallas as pl
from jax.experimental.pallas import tpu as pltpu
from jax.experimental.pallas import tpu_sc as plsc

NC, NS = 2, 16
NW = NC * NS
LN = 128
JC = 25


@functools.lru_cache(maxsize=None)
def _make_kernel(nrows: int, ncols: int, vocab: int, embed_dim: int):
    mesh = plsc.VectorSubcoreMesh(core_axis_name="c", subcore_axis_name="s")
    nit = nrows // LN
    nit_w = nit // NW
    njc = ncols // JC
    kt = embed_dim // 8

    @functools.partial(
        pl.kernel,
        out_type=jax.ShapeDtypeStruct((ncols, kt, nit, 8, LN), jnp.float32),
        mesh=mesh,
        compiler_params=pltpu.CompilerParams(
            use_tc_tiling_on_sc=False, needs_layout_passes=False
        ),
        scratch_types=[
            pltpu.VMEM((embed_dim * vocab * 16,), jnp.float32),
            pltpu.VMEM((2, JC, LN), jnp.int32),
            pltpu.VMEM((2, JC, kt, 8, LN), jnp.float32),
            pltpu.SemaphoreType.DMA,
            pltpu.SemaphoreType.DMA,
        ],
    )
    def kern(ids_t_hbm, table_hbm, out_hbm, table_v, idx_v, out_v, sem_o, sem_i):
        cid = lax.axis_index("c")
        sid = lax.axis_index("s")
        wid = sid * NC + cid
        nchunks = nit_w * njc

        pltpu.sync_copy(table_hbm, table_v)

        lane = lax.iota(jnp.int32, 16)

        def out_slice(c):
            it = wid * nit_w + c // njc
            j0 = (c % njc) * JC
            return out_hbm.at[pl.ds(j0, JC), :, it]

        def ids_slice(c):
            cc = jnp.minimum(c, nchunks - 1)
            it = wid * nit_w + cc // njc
            j0 = (cc % njc) * JC
            return ids_t_hbm.at[pl.ds(j0, JC), pl.ds(it * LN, LN)]

        def idx_start(c, b):
            pltpu.async_copy(ids_slice(c), idx_v.at[b], sem_i)

        def idx_wait(c, b):
            pltpu.make_async_copy(ids_slice(c), idx_v.at[b], sem_i).wait()

        def compute_chunk(c, b):
            @plsc.parallel_loop(0, JC, unroll=4)
            def j_loop(jj):
                for g0 in range(0, LN // 16, 2):
                    bases = [
                        idx_v[b, jj, pl.ds((g0 + d) * 16, 16)] * 16 + lane
                        for d in range(2)
                    ]
                    for k in range(embed_dim):
                        for d in range(2):
                            v = plsc.load_gather(
                                table_v, [bases[d] + k * vocab * 16]
                            )
                            out_v[
                                b, jj, k // 8, k % 8, pl.ds((g0 + d) * 16, 16)
                            ] = v

        def store_start(c, b):
            pltpu.async_copy(out_v.at[b], out_slice(c), sem_o)

        def store_wait(c, b):
            pltpu.make_async_copy(out_v.at[b], out_slice(c), sem_o).wait()

        idx_start(0, 0)
        idx_wait(0, 0)
        idx_start(1, 1)
        compute_chunk(0, 0)
        store_start(0, 0)
        idx_wait(1, 1)
        idx_start(2, 0)
        compute_chunk(1, 1)
        store_start(1, 1)

        @pl.loop(2, nchunks, step=2)
        def chunk_loop(c0):
            for b in range(2):
                c = c0 + b
                idx_wait(c, b)

                @pl.when(c + 1 < nchunks)
                def _():
                    idx_start(c + 1, 1 - b)

                store_wait(c - 2, b)
                compute_chunk(c, b)
                store_start(c, b)

        store_wait(nchunks - 2, 0)
        store_wait(nchunks - 1, 1)

    return kern


def kernel(ids, embed):
    nrows, ncols = ids.shape
    vocab, embed_dim = embed.shape
    ids_t = jnp.transpose(ids.astype(jnp.int32))
    table_rep = jnp.broadcast_to(
        jnp.transpose(embed)[:, :, None], (embed_dim, vocab, 16)
    ).reshape(-1)
    a = _make_kernel(nrows, ncols, vocab, embed_dim)(ids_t, table_rep)
    return a.transpose(2, 4, 0, 1, 3).reshape(nrows, ncols, embed_dim)


# speedup vs baseline: 2.8486x; 1.0505x over previous
import functools

import jax
import jax.numpy as jnp
from jax import lax
from jax.experimental import pallas as pl
from jax.experimental.pallas import tpu as pltpu
from jax.experimental.pallas import tpu_sc as plsc

NC, NS = 2, 16
NW = NC * NS
LN = 128
JC = 25


@functools.lru_cache(maxsize=None)
def _make_kernel(nrows: int, ncols: int, vocab: int, embed_dim: int):
    mesh = plsc.VectorSubcoreMesh(core_axis_name="c", subcore_axis_name="s")
    nit = nrows // LN
    nit_w = nit // NW
    njc = ncols // JC
    kt = embed_dim // 8

    @functools.partial(
        pl.kernel,
        out_type=jax.ShapeDtypeStruct((ncols, kt, nit, 8, LN), jnp.float32),
        mesh=mesh,
        compiler_params=pltpu.CompilerParams(
            use_tc_tiling_on_sc=False, needs_layout_passes=False
        ),
        scratch_types=[
            pltpu.VMEM((embed_dim * vocab * 16,), jnp.float32),
            pltpu.VMEM((2, JC, LN), jnp.int32),
            pltpu.VMEM((2, JC, kt, 8, LN), jnp.float32),
            pltpu.SemaphoreType.DMA,
            pltpu.SemaphoreType.DMA,
        ],
    )
    def kern(ids_t_hbm, table_hbm, out_hbm, table_v, idx_v, out_v, sem_o, sem_i):
        cid = lax.axis_index("c")
        sid = lax.axis_index("s")
        wid = sid * NC + cid
        nchunks = nit_w * njc

        pltpu.sync_copy(table_hbm, table_v)

        lane = lax.iota(jnp.int32, 16)

        def out_slice(c):
            it = wid * nit_w + c // njc
            j0 = (c % njc) * JC
            return out_hbm.at[pl.ds(j0, JC), :, it]

        def ids_slice(c):
            cc = jnp.minimum(c, nchunks - 1)
            it = wid * nit_w + cc // njc
            j0 = (cc % njc) * JC
            return ids_t_hbm.at[pl.ds(j0, JC), pl.ds(it * LN, LN)]

        def idx_start(c, b):
            pltpu.async_copy(ids_slice(c), idx_v.at[b], sem_i)

        def idx_wait(c, b):
            pltpu.make_async_copy(ids_slice(c), idx_v.at[b], sem_i).wait()

        def compute_chunk(c, b):
            @plsc.parallel_loop(0, JC, unroll=2)
            def j_loop(jj):
                for g0 in range(0, LN // 16, 4):
                    bases = [
                        idx_v[b, jj, pl.ds((g0 + d) * 16, 16)] * 16 + lane
                        for d in range(4)
                    ]
                    for k in range(embed_dim):
                        for d in range(4):
                            v = plsc.load_gather(
                                table_v, [bases[d] + k * vocab * 16]
                            )
                            out_v[
                                b, jj, k // 8, k % 8, pl.ds((g0 + d) * 16, 16)
                            ] = v

        def store_start(c, b):
            pltpu.async_copy(out_v.at[b], out_slice(c), sem_o)

        def store_wait(c, b):
            pltpu.make_async_copy(out_v.at[b], out_slice(c), sem_o).wait()

        idx_start(0, 0)
        idx_wait(0, 0)
        idx_start(1, 1)
        compute_chunk(0, 0)
        store_start(0, 0)
        idx_wait(1, 1)
        idx_start(2, 0)
        compute_chunk(1, 1)
        store_start(1, 1)

        @pl.loop(2, nchunks, step=2)
        def chunk_loop(c0):
            for b in range(2):
                c = c0 + b
                idx_wait(c, b)

                @pl.when(c + 1 < nchunks)
                def _():
                    idx_start(c + 1, 1 - b)

                store_wait(c - 2, b)
                compute_chunk(c, b)
                store_start(c, b)

        store_wait(nchunks - 2, 0)
        store_wait(nchunks - 1, 1)

    return kern


def kernel(ids, embed):
    nrows, ncols = ids.shape
    vocab, embed_dim = embed.shape
    ids_t = jnp.transpose(ids.astype(jnp.int32))
    table_rep = jnp.broadcast_to(
        jnp.transpose(embed)[:, :, None], (embed_dim, vocab, 16)
    ).reshape(-1)
    a = _make_kernel(nrows, ncols, vocab, embed_dim)(ids_t, table_rep)
    return a.transpose(2, 4, 0, 1, 3).reshape(nrows, ncols, embed_dim)
